# baseline (speedup 1.0000x reference)
import functools

import jax
import jax.numpy as jnp
from jax import lax
from jax.experimental import pallas as pl
from jax.experimental.pallas import tpu as pltpu
from jax.experimental.pallas import tpu_sc as plsc

D = 32
IPG = 128
BT_PER_TASK = 4
TASK = BT_PER_TASK * IPG


@functools.cache
def _build(l_dim, b_dim, nc, ns):
    total = l_dim * b_dim
    nw = nc * ns
    ntask = total // TASK
    tpw = ntask // nw
    assert tpw % 2 == 0 and tpw >= 4 and ntask * TASK == total
    g_rows = total // IPG
    out_rows = total * D // (8 * IPG)

    mesh = plsc.VectorSubcoreMesh(
        core_axis_name="c", subcore_axis_name="s",
        num_cores=nc, num_subcores=ns)

    @functools.partial(
        pl.kernel,
        out_type=jax.ShapeDtypeStruct((out_rows, 8, IPG), jnp.float32),
        mesh=mesh,
        scratch_types=[
            pltpu.VMEM((2, BT_PER_TASK, IPG), jnp.int32),
            pltpu.VMEM((2, TASK, D), jnp.float32),
            pltpu.VMEM((2, D // 8, BT_PER_TASK, 8, IPG), jnp.float32),
            pltpu.SemaphoreType.DMA,
            pltpu.SemaphoreType.DMA,
            pltpu.SemaphoreType.DMA,
            pltpu.SemaphoreType.DMA,
            pltpu.SemaphoreType.DMA,
            pltpu.SemaphoreType.DMA,
        ],
        compiler_params=pltpu.CompilerParams(
            use_tc_tiling_on_sc=False, needs_layout_passes=False),
    )
    def gather_kernel(table_hbm, idx_hbm, out_hbm, idx_v, rows_v, stg_v,
                      isem0, isem1, gsem0, gsem1, osem0, osem1):
        wid = lax.axis_index("s") * nc + lax.axis_index("c")
        t0 = wid * tpw
        isem = (isem0, isem1)
        gsem = (gsem0, gsem1)
        osem = (osem0, osem1)
        lanes = lax.iota(jnp.int32, 16)

        def s_idx(t, bf):
            pltpu.make_async_copy(
                idx_hbm.at[pl.ds(t * BT_PER_TASK, BT_PER_TASK)],
                idx_v.at[bf], isem[bf],
            ).start()

        def w_idx(bf):
            pltpu.make_async_copy(
                idx_hbm.at[pl.ds(0, BT_PER_TASK)], idx_v.at[bf], isem[bf]
            ).wait()

        def g_fire(bf):
            for j in range(BT_PER_TASK):
                pltpu.make_async_copy(
                    table_hbm.at[idx_v.at[bf, j]],
                    rows_v.at[bf, pl.ds(j * IPG, IPG)],
                    gsem[bf],
                ).start()

        def g_drain(bf):
            for j in range(BT_PER_TASK):
                pltpu.make_async_copy(
                    table_hbm.at[idx_v.at[bf, j]],
                    rows_v.at[bf, pl.ds(j * IPG, IPG)],
                    gsem[bf],
                ).wait()

        def transpose(bf):
            src = rows_v.at[bf]

            @plsc.parallel_loop(0, (D // 8) * BT_PER_TASK, 1, unroll=4)
            def _(k):
                dt = k // BT_PER_TASK
                btl = k % BT_PER_TASK
                for dr in range(8):
                    d = dt * 8 + dr
                    col = jnp.broadcast_to(d, (16,)).astype(jnp.int32)
                    for q in range(IPG // 16):
                        row = btl * IPG + q * 16 + lanes
                        v = plsc.load_gather(src, [row, col])
                        stg_v[bf, dt, btl, dr, pl.ds(q * 16, 16)] = v

        def s_out(t, bf):
            l = t // (b_dim // TASK)
            btg = t % (b_dim // TASK)
            for dt in range(D // 8):
                row0 = l * (b_dim // IPG) * (D // 8) + dt * (b_dim // IPG) \
                    + btg * BT_PER_TASK
                pltpu.make_async_copy(
                    stg_v.at[bf, dt],
                    out_hbm.at[pl.ds(row0, BT_PER_TASK)],
                    osem[bf],
                ).start()

        def w_out(bf):
            for dt in range(D // 8):
                pltpu.make_async_copy(
                    stg_v.at[bf, dt], out_hbm.at[pl.ds(0, BT_PER_TASK)],
                    osem[bf],
                ).wait()

        s_idx(t0, 0)
        s_idx(t0 + 1, 1)
        w_idx(0)
        g_fire(0)

        def step(i, bf):
            t = t0 + i
            g_drain(bf)

            @pl.when(i + 1 < tpw)
            def _():
                w_idx(1 - bf)
                g_fire(1 - bf)

            @pl.when(i + 2 < tpw)
            def _():
                s_idx(t + 2, bf)

            @pl.when(i >= 2)
            def _():
                w_out(bf)

            transpose(bf)
            s_out(t, bf)

        def pair(p, carry):
            step(2 * p, 0)
            step(2 * p + 1, 1)
            return carry

        lax.fori_loop(0, tpw // 2, pair, 0)
        w_out(0)
        w_out(1)

    return gather_kernel


def kernel(actions, table):
    b, l = actions.shape
    info = plsc.get_sparse_core_info()
    nc, ns = info.num_cores, info.num_subcores
    idx2d = actions.astype(jnp.int32).T.reshape((b * l) // IPG, IPG)
    res = _build(l, b, nc, ns)(table, idx2d)
    out5 = res.reshape(l, D // 8, b // IPG, 8, IPG)
    return out5.transpose(2, 4, 0, 1, 3).reshape(b, l, 1, D)


# speedup vs baseline: 1.1479x; 1.1479x over previous
import functools

import jax
import jax.numpy as jnp
from jax import lax
from jax.experimental import pallas as pl
from jax.experimental.pallas import tpu as pltpu
from jax.experimental.pallas import tpu_sc as plsc

D = 32
IPG = 128
BT_PER_TASK = 4
TASK = BT_PER_TASK * IPG


@functools.cache
def _build(l_dim, b_dim, nc, ns):
    total = l_dim * b_dim
    nw = nc * ns
    ntask = total // TASK
    tpw = ntask // nw
    assert tpw % 2 == 0 and tpw >= 4 and ntask * TASK == total
    g_rows = total // IPG
    out_rows = total * D // (8 * IPG)

    mesh = plsc.VectorSubcoreMesh(
        core_axis_name="c", subcore_axis_name="s",
        num_cores=nc, num_subcores=ns)

    @functools.partial(
        pl.kernel,
        out_type=jax.ShapeDtypeStruct((out_rows * 8 * IPG,), jnp.float32),
        mesh=mesh,
        scratch_types=[
            pltpu.VMEM((2, BT_PER_TASK, IPG), jnp.int32),
            pltpu.VMEM((2, TASK, D), jnp.float32),
            pltpu.VMEM((2, TASK * D), jnp.float32),
            pltpu.SemaphoreType.DMA,
            pltpu.SemaphoreType.DMA,
            pltpu.SemaphoreType.DMA,
            pltpu.SemaphoreType.DMA,
            pltpu.SemaphoreType.DMA,
            pltpu.SemaphoreType.DMA,
        ],
        compiler_params=pltpu.CompilerParams(
            use_tc_tiling_on_sc=False, needs_layout_passes=False),
    )
    def gather_kernel(table_hbm, idx_hbm, out_hbm, idx_v, rows_v, stg_v,
                      isem0, isem1, gsem0, gsem1, osem0, osem1):
        wid = lax.axis_index("s") * nc + lax.axis_index("c")
        t0 = wid * tpw
        isem = (isem0, isem1)
        gsem = (gsem0, gsem1)
        osem = (osem0, osem1)
        lanes = lax.iota(jnp.int32, 16)

        def s_idx(t, bf):
            pltpu.make_async_copy(
                idx_hbm.at[pl.ds(t * BT_PER_TASK, BT_PER_TASK)],
                idx_v.at[bf], isem[bf],
            ).start()

        def w_idx(bf):
            pltpu.make_async_copy(
                idx_hbm.at[pl.ds(0, BT_PER_TASK)], idx_v.at[bf], isem[bf]
            ).wait()

        def g_fire(bf):
            for j in range(BT_PER_TASK):
                pltpu.make_async_copy(
                    table_hbm.at[idx_v.at[bf, j]],
                    rows_v.at[bf, pl.ds(j * IPG, IPG)],
                    gsem[bf],
                ).start()

        def g_drain(bf):
            for j in range(BT_PER_TASK):
                pltpu.make_async_copy(
                    table_hbm.at[idx_v.at[bf, j]],
                    rows_v.at[bf, pl.ds(j * IPG, IPG)],
                    gsem[bf],
                ).wait()

        vbase = (lanes >> 3) * 4096 + (lanes & 7) * IPG

        def transpose(bf):
            stg = stg_v.at[bf]

            @plsc.parallel_loop(0, TASK, 1, unroll=4)
            def _(r):
                off = (r // IPG) * 1024 + (r % IPG)
                idx0 = vbase + off
                v0 = rows_v[bf, r, pl.ds(0, 16)]
                v1 = rows_v[bf, r, pl.ds(16, 16)]
                plsc.store_scatter(stg, [idx0], v0)
                plsc.store_scatter(stg, [idx0 + 2 * 4096], v1)

        def s_out(t, bf):
            l = t // (b_dim // TASK)
            btg = t % (b_dim // TASK)
            for dt in range(D // 8):
                row0 = l * (b_dim // IPG) * (D // 8) + dt * (b_dim // IPG) \
                    + btg * BT_PER_TASK
                pltpu.make_async_copy(
                    stg_v.at[bf, pl.ds(dt * 4096, 4096)],
                    out_hbm.at[pl.ds(row0 * 1024, 4096)],
                    osem[bf],
                ).start()

        def w_out(bf):
            for dt in range(D // 8):
                pltpu.make_async_copy(
                    stg_v.at[bf, pl.ds(dt * 4096, 4096)],
                    out_hbm.at[pl.ds(0, 4096)],
                    osem[bf],
                ).wait()

        s_idx(t0, 0)
        s_idx(t0 + 1, 1)
        w_idx(0)
        g_fire(0)

        def step(i, bf):
            t = t0 + i
            g_drain(bf)

            @pl.when(i + 1 < tpw)
            def _():
                w_idx(1 - bf)
                g_fire(1 - bf)

            @pl.when(i + 2 < tpw)
            def _():
                s_idx(t + 2, bf)

            @pl.when(i >= 2)
            def _():
                w_out(bf)

            transpose(bf)
            s_out(t, bf)

        def pair(p, carry):
            step(2 * p, 0)
            step(2 * p + 1, 1)
            return carry

        lax.fori_loop(0, tpw // 2, pair, 0)
        w_out(0)
        w_out(1)

    return gather_kernel


def kernel(actions, table):
    b, l = actions.shape
    info = plsc.get_sparse_core_info()
    nc, ns = info.num_cores, info.num_subcores
    idx2d = actions.astype(jnp.int32).T.reshape((b * l) // IPG, IPG)
    res = _build(l, b, nc, ns)(table, idx2d)
    out5 = res.reshape(l, D // 8, b // IPG, 8, IPG)
    return out5.transpose(2, 4, 0, 1, 3).reshape(b, l, 1, D)


# speedup vs baseline: 1.1482x; 1.0002x over previous
import functools

import jax
import jax.numpy as jnp
from jax import lax
from jax.experimental import pallas as pl
from jax.experimental.pallas import tpu as pltpu
from jax.experimental.pallas import tpu_sc as plsc

D = 32
IPG = 128
BT_PER_TASK = 4
TASK = BT_PER_TASK * IPG


@functools.cache
def _build(l_dim, b_dim, nc, ns):
    total = l_dim * b_dim
    nw = nc * ns
    ntask = total // TASK
    tpw = ntask // nw
    assert tpw % 2 == 0 and tpw >= 4 and ntask * TASK == total
    g_rows = total // IPG
    out_rows = total * D // (8 * IPG)

    mesh = plsc.VectorSubcoreMesh(
        core_axis_name="c", subcore_axis_name="s",
        num_cores=nc, num_subcores=ns)

    @functools.partial(
        pl.kernel,
        out_type=jax.ShapeDtypeStruct((out_rows * 8 * IPG,), jnp.float32),
        mesh=mesh,
        scratch_types=[
            pltpu.VMEM((2, BT_PER_TASK, IPG), jnp.int32),
            pltpu.VMEM((2, TASK, D), jnp.float32),
            pltpu.VMEM((2, TASK * D), jnp.float32),
            pltpu.SemaphoreType.DMA,
            pltpu.SemaphoreType.DMA,
            pltpu.SemaphoreType.DMA,
            pltpu.SemaphoreType.DMA,
            pltpu.SemaphoreType.DMA,
            pltpu.SemaphoreType.DMA,
        ],
        compiler_params=pltpu.CompilerParams(
            use_tc_tiling_on_sc=False, needs_layout_passes=False),
    )
    def gather_kernel(table_hbm, idx_hbm, out_hbm, idx_v, rows_v, stg_v,
                      isem0, isem1, gsem0, gsem1, osem0, osem1):
        wid = lax.axis_index("s") * nc + lax.axis_index("c")
        t0 = wid * tpw
        isem = (isem0, isem1)
        gsem = (gsem0, gsem1)
        osem = (osem0, osem1)
        lanes = lax.iota(jnp.int32, 16)

        def s_idx(t, bf):
            pltpu.make_async_copy(
                idx_hbm.at[pl.ds(t * BT_PER_TASK, BT_PER_TASK)],
                idx_v.at[bf], isem[bf],
            ).start()

        def w_idx(bf):
            pltpu.make_async_copy(
                idx_hbm.at[pl.ds(0, BT_PER_TASK)], idx_v.at[bf], isem[bf]
            ).wait()

        def g_fire(bf):
            for j in range(BT_PER_TASK):
                pltpu.make_async_copy(
                    table_hbm.at[idx_v.at[bf, j]],
                    rows_v.at[bf, pl.ds(j * IPG, IPG)],
                    gsem[bf],
                ).start()

        def g_drain(bf):
            for j in range(BT_PER_TASK):
                pltpu.make_async_copy(
                    table_hbm.at[idx_v.at[bf, j]],
                    rows_v.at[bf, pl.ds(j * IPG, IPG)],
                    gsem[bf],
                ).wait()

        vbase = (lanes >> 3) * 4096 + (lanes & 7) * IPG

        def transpose(bf):
            stg = stg_v.at[bf]

            @plsc.parallel_loop(0, TASK, 1, unroll=4)
            def _(r):
                off = ((r >> 7) << 10) + (r & (IPG - 1))
                idx0 = vbase + off
                v0 = rows_v[bf, r, pl.ds(0, 16)]
                v1 = rows_v[bf, r, pl.ds(16, 16)]
                plsc.store_scatter(stg, [idx0], v0)
                plsc.store_scatter(stg, [idx0 + 2 * 4096], v1)

        def s_out(t, bf):
            l = t // (b_dim // TASK)
            btg = t % (b_dim // TASK)
            for dt in range(D // 8):
                row0 = l * (b_dim // IPG) * (D // 8) + dt * (b_dim // IPG) \
                    + btg * BT_PER_TASK
                pltpu.make_async_copy(
                    stg_v.at[bf, pl.ds(dt * 4096, 4096)],
                    out_hbm.at[pl.ds(row0 * 1024, 4096)],
                    osem[bf],
                ).start()

        def w_out(bf):
            for dt in range(D // 8):
                pltpu.make_async_copy(
                    stg_v.at[bf, pl.ds(dt * 4096, 4096)],
                    out_hbm.at[pl.ds(0, 4096)],
                    osem[bf],
                ).wait()

        s_idx(t0, 0)
        s_idx(t0 + 1, 1)
        w_idx(0)
        g_fire(0)

        def step(i, bf):
            t = t0 + i
            g_drain(bf)

            @pl.when(i + 1 < tpw)
            def _():
                w_idx(1 - bf)
                g_fire(1 - bf)

            @pl.when(i + 2 < tpw)
            def _():
                s_idx(t + 2, bf)

            @pl.when(i >= 2)
            def _():
                w_out(bf)

            transpose(bf)
            s_out(t, bf)

        def pair(p, carry):
            step(2 * p, 0)
            step(2 * p + 1, 1)
            return carry

        lax.fori_loop(0, tpw // 2, pair, 0)
        w_out(0)
        w_out(1)

    return gather_kernel


def kernel(actions, table):
    b, l = actions.shape
    info = plsc.get_sparse_core_info()
    nc, ns = info.num_cores, info.num_subcores
    idx2d = actions.astype(jnp.int32).T.reshape((b * l) // IPG, IPG)
    res = _build(l, b, nc, ns)(table, idx2d)
    out5 = res.reshape(l, D // 8, b // IPG, 8, IPG)
    return out5.transpose(2, 4, 0, 1, 3).reshape(b, l, 1, D)


# speedup vs baseline: 3.5794x; 3.1175x over previous
import functools

import jax
import jax.numpy as jnp
from jax import lax
from jax.experimental import pallas as pl
from jax.experimental.pallas import tpu as pltpu
from jax.experimental.pallas import tpu_sc as plsc

D = 32
IPG = 128
BT_PER_TASK = 4
TASK = BT_PER_TASK * IPG


@functools.cache
def _build(l_dim, b_dim, nc, ns):
    total = l_dim * b_dim
    nw = nc * ns
    ntask = total // TASK
    tpw = ntask // nw
    assert tpw % 2 == 0 and tpw >= 4 and ntask * TASK == total
    g_rows = total // IPG
    out_rows = total * D // (8 * IPG)

    mesh = plsc.VectorSubcoreMesh(
        core_axis_name="c", subcore_axis_name="s",
        num_cores=nc, num_subcores=ns)

    @functools.partial(
        pl.kernel,
        out_type=jax.ShapeDtypeStruct((out_rows * 8 * IPG,), jnp.float32),
        mesh=mesh,
        scratch_types=[
            pltpu.VMEM((2, BT_PER_TASK, IPG), jnp.int32),
            pltpu.VMEM((2, TASK, D + 1), jnp.float32),
            pltpu.VMEM((2, TASK * D), jnp.float32),
            pltpu.SemaphoreType.DMA,
            pltpu.SemaphoreType.DMA,
            pltpu.SemaphoreType.DMA,
            pltpu.SemaphoreType.DMA,
            pltpu.SemaphoreType.DMA,
            pltpu.SemaphoreType.DMA,
        ],
        compiler_params=pltpu.CompilerParams(
            use_tc_tiling_on_sc=False, needs_layout_passes=False),
    )
    def gather_kernel(table_hbm, idx_hbm, out_hbm, idx_v, rows_v, stg_v,
                      isem0, isem1, gsem0, gsem1, osem0, osem1):
        wid = lax.axis_index("s") * nc + lax.axis_index("c")
        t0 = wid * tpw
        isem = (isem0, isem1)
        gsem = (gsem0, gsem1)
        osem = (osem0, osem1)
        lanes = lax.iota(jnp.int32, 16)

        def s_idx(t, bf):
            pltpu.make_async_copy(
                idx_hbm.at[pl.ds(t * BT_PER_TASK, BT_PER_TASK)],
                idx_v.at[bf], isem[bf],
            ).start()

        def w_idx(bf):
            pltpu.make_async_copy(
                idx_hbm.at[pl.ds(0, BT_PER_TASK)], idx_v.at[bf], isem[bf]
            ).wait()

        def g_fire(bf):
            for j in range(BT_PER_TASK):
                pltpu.make_async_copy(
                    table_hbm.at[idx_v.at[bf, j]],
                    rows_v.at[bf, pl.ds(j * IPG, IPG)],
                    gsem[bf],
                ).start()

        def g_drain(bf):
            for j in range(BT_PER_TASK):
                pltpu.make_async_copy(
                    table_hbm.at[idx_v.at[bf, j]],
                    rows_v.at[bf, pl.ds(j * IPG, IPG)],
                    gsem[bf],
                ).wait()

        cols = [jnp.broadcast_to(jnp.int32(d), (16,)) for d in range(D)]

        def transpose(bf):
            src = rows_v.at[bf]

            @plsc.parallel_loop(0, TASK // 16, 1, unroll=2)
            def _(k):
                btl = k >> 3
                q = k & 7
                rows = (btl * IPG + q * 16) + lanes
                base = btl * 1024 + q * 16
                for dt in range(D // 8):
                    for dr in range(8):
                        v = plsc.load_gather(src, [rows, cols[dt * 8 + dr]])
                        stg_v[bf, pl.ds(base + dt * 4096 + dr * IPG, 16)] = v

        def s_out(t, bf):
            l = t // (b_dim // TASK)
            btg = t % (b_dim // TASK)
            for dt in range(D // 8):
                row0 = l * (b_dim // IPG) * (D // 8) + dt * (b_dim // IPG) \
                    + btg * BT_PER_TASK
                pltpu.make_async_copy(
                    stg_v.at[bf, pl.ds(dt * 4096, 4096)],
                    out_hbm.at[pl.ds(row0 * 1024, 4096)],
                    osem[bf],
                ).start()

        def w_out(bf):
            for dt in range(D // 8):
                pltpu.make_async_copy(
                    stg_v.at[bf, pl.ds(dt * 4096, 4096)],
                    out_hbm.at[pl.ds(0, 4096)],
                    osem[bf],
                ).wait()

        s_idx(t0, 0)
        s_idx(t0 + 1, 1)
        w_idx(0)
        g_fire(0)

        def step(i, bf):
            t = t0 + i
            g_drain(bf)

            @pl.when(i + 1 < tpw)
            def _():
                w_idx(1 - bf)
                g_fire(1 - bf)

            @pl.when(i + 2 < tpw)
            def _():
                s_idx(t + 2, bf)

            @pl.when(i >= 2)
            def _():
                w_out(bf)

            transpose(bf)
            s_out(t, bf)

        def pair(p, carry):
            step(2 * p, 0)
            step(2 * p + 1, 1)
            return carry

        lax.fori_loop(0, tpw // 2, pair, 0)
        w_out(0)
        w_out(1)

    return gather_kernel


def kernel(actions, table):
    b, l = actions.shape
    info = plsc.get_sparse_core_info()
    nc, ns = info.num_cores, info.num_subcores
    idx2d = actions.astype(jnp.int32).T.reshape((b * l) // IPG, IPG)
    table_p = jnp.pad(table, ((0, 0), (0, 1)))
    res = _build(l, b, nc, ns)(table_p, idx2d)
    out5 = res.reshape(l, D // 8, b // IPG, 8, IPG)
    return out5.transpose(2, 4, 0, 1, 3).reshape(b, l, 1, D)
